# baseline (speedup 1.0000x reference)
import functools

import jax
import jax.numpy as jnp
from jax import lax
from jax.experimental import pallas as pl
from jax.experimental.pallas import tpu as pltpu
from jax.experimental.pallas import tpu_sc as plsc

N_NODES = 50000
N_EDGES = 800000
P = 8
D = 4 * P
NC = 2
NS = 16
NW = NC * NS
L = 16

U = 256
G = U // L
GB = 128
NUNITS = N_EDGES // U
MAIN = NUNITS // NW
EXTRA_W = NUNITS - MAIN * NW
NCH = MAIN + 1


def _splat(v):
    return jnp.full((L,), v, dtype=jnp.int32)


def _i32(v):
    return jnp.int32(v)


def _fori(n, body):
    lax.fori_loop(_i32(0), _i32(n), body, _i32(0))


def _sc_body(ptab, ec, wts, ratios, rmw,
             ecv, eiv, ejv, xiv, xjv, outv, wv, w8v,
             sem_idx, sem_g, sem_wb):
    wid = lax.axis_index("s") * NC + lax.axis_index("c")
    start_u = wid * _i32(MAIN) + jnp.minimum(wid, _i32(EXTRA_W))
    last_c = jnp.where(wid < _i32(EXTRA_W), _i32(MAIN), _i32(MAIN - 1))
    iota16 = lax.iota(jnp.int32, L)
    zero16 = jnp.zeros((L,), dtype=jnp.int32)
    c0s, c1s = _splat(0), _splat(1)

    def ebase(c):
        return (start_u + jnp.minimum(c, last_c)) * _i32(U)

    def idx_issue(c, b):
        bo = b * _i32(U)
        pltpu.async_copy(ec.at[pl.ds(ebase(c), U)], ecv.at[pl.ds(bo, U)],
                         sem_idx.at[b])

    def idx_wait(b):
        bo = b * _i32(U)
        pltpu.make_async_copy(ec.at[pl.ds(0, U)], ecv.at[pl.ds(bo, U)],
                              sem_idx.at[b]).wait()

    def dec(b):
        bo16 = jnp.full((L,), b * _i32(U), dtype=jnp.int32)
        bo = b * _i32(U)

        @plsc.parallel_loop(_i32(0), _i32(G), step=_i32(1))
        def _(g):
            o = g * _i32(L)
            e16 = bo16 + o + iota16
            eiv[pl.ds(bo + o, L)] = plsc.load_gather(ecv, [e16, c0s])
            ejv[pl.ds(bo + o, L)] = plsc.load_gather(ecv, [e16, c1s])

    def gath_issue(b):
        bo = b * _i32(U)
        for s in (0, GB):
            pltpu.async_copy(ptab.at[eiv.at[pl.ds(bo + s, GB)]],
                             xiv.at[pl.ds(bo + s, GB)], sem_g.at[b])
            pltpu.async_copy(ptab.at[ejv.at[pl.ds(bo + s, GB)]],
                             xjv.at[pl.ds(bo + s, GB)], sem_g.at[b])

    def gath_wait(b):
        bo = b * _i32(U)
        for s in (0, GB):
            pltpu.make_async_copy(ptab.at[eiv.at[pl.ds(bo + s, GB)]],
                                  xiv.at[pl.ds(bo + s, GB)],
                                  sem_g.at[b]).wait()
            pltpu.make_async_copy(ptab.at[ejv.at[pl.ds(bo + s, GB)]],
                                  xjv.at[pl.ds(bo + s, GB)],
                                  sem_g.at[b]).wait()

    def comp(b):
        bo16 = jnp.full((L,), b * _i32(U), dtype=jnp.int32)

        @plsc.parallel_loop(_i32(0), _i32(G), step=_i32(1))
        def _(g):
            e16 = bo16 + g * _i32(L) + iota16
            for p in range(P):
                q = 4 * p
                w1 = plsc.load_gather(xiv, [e16, _splat(q)])
                x1 = plsc.load_gather(xiv, [e16, _splat(q + 1)])
                y1 = plsc.load_gather(xiv, [e16, _splat(q + 2)])
                z1 = plsc.load_gather(xiv, [e16, _splat(q + 3)])
                w2 = plsc.load_gather(xjv, [e16, _splat(q)])
                x2 = plsc.load_gather(xjv, [e16, _splat(q + 1)])
                y2 = plsc.load_gather(xjv, [e16, _splat(q + 2)])
                z2 = plsc.load_gather(xjv, [e16, _splat(q + 3)])
                rw = (w1 * w2 + x1 * x2) + (y1 * y2 + z1 * z2)
                rx = (x1 * w2 - w1 * x2) + (z1 * y2 - y1 * z2)
                ry = (y1 * w2 - w1 * y2) + (x1 * z2 - z1 * x2)
                rz = (z1 * w2 - w1 * z2) + (y1 * x2 - x1 * y2)
                plsc.store_scatter(outv, [e16, _splat(q)], rw)
                plsc.store_scatter(outv, [e16, _splat(q + 1)], rx)
                plsc.store_scatter(outv, [e16, _splat(q + 2)], ry)
                plsc.store_scatter(outv, [e16, _splat(q + 3)], rz)

    def wb_issue(c, b):
        bb = ebase(c)
        bo = b * _i32(U)
        pltpu.async_copy(outv.at[pl.ds(bo, U)], ratios.at[pl.ds(bb, U)],
                         sem_wb.at[b])
        pltpu.async_copy(wv, rmw.at[pl.ds(bb * _i32(P), U * P)], sem_wb.at[b])

    def wb_wait(b):
        bo = b * _i32(U)
        pltpu.make_async_copy(ratios.at[pl.ds(0, U)], outv.at[pl.ds(bo, U)],
                              sem_wb.at[b]).wait()
        pltpu.make_async_copy(rmw.at[pl.ds(0, U * P)], wv, sem_wb.at[b]).wait()

    pltpu.sync_copy(wts, w8v)
    wvals = plsc.load_gather(w8v, [zero16, iota16 & _i32(7)])

    def wfill(s, carry):
        wv[pl.ds(s * _i32(L), L)] = wvals
        return carry
    _fori(U * P // L, wfill)

    pltpu.sync_copy(ec.at[pl.ds(ebase(_i32(0)), U)], ecv.at[pl.ds(0, U)])
    dec(_i32(0))
    gath_issue(_i32(0))
    idx_issue(_i32(1), _i32(1))

    def chunk(k, carry):
        par = k & _i32(1)
        nxt = _i32(1) - par

        @pl.when(k + _i32(2) < _i32(NCH))
        def _():
            idx_issue(k + _i32(2), par)

        @pl.when(k + _i32(1) < _i32(NCH))
        def _():
            idx_wait(nxt)
            dec(nxt)
            gath_issue(nxt)

        @pl.when(k >= _i32(2))
        def _():
            wb_wait(par)

        gath_wait(par)
        comp(par)
        wb_issue(k, par)
        return carry
    _fori(NCH, chunk)

    wb_wait(_i32(0))
    wb_wait(_i32(1))


@functools.partial(
    pl.kernel,
    out_type=(jax.ShapeDtypeStruct((N_EDGES, D), jnp.float32),
              jax.ShapeDtypeStruct((N_EDGES * P,), jnp.float32)),
    mesh=plsc.VectorSubcoreMesh(core_axis_name="c", subcore_axis_name="s",
                                num_cores=NC, num_subcores=NS),
    compiler_params=pltpu.CompilerParams(needs_layout_passes=False,
                                         use_tc_tiling_on_sc=False),
    scratch_types=[
        pltpu.VMEM((2 * U, 2), jnp.int32),
        pltpu.VMEM((2 * U,), jnp.int32),
        pltpu.VMEM((2 * U,), jnp.int32),
        pltpu.VMEM((2 * U, D), jnp.float32),
        pltpu.VMEM((2 * U, D), jnp.float32),
        pltpu.VMEM((2 * U, D), jnp.float32),
        pltpu.VMEM((U * P,), jnp.float32),
        pltpu.VMEM((1, P), jnp.float32),
        pltpu.SemaphoreType.DMA((2,)),
        pltpu.SemaphoreType.DMA((2,)),
        pltpu.SemaphoreType.DMA((2,)),
    ],
)
def _quat_edges_sc(ptab, ec, wts, ratios, rmw, *scratch):
    _sc_body(ptab, ec, wts, ratios, rmw, *scratch)


def kernel(particles, weights, edges):
    ec = edges.astype(jnp.int32)
    ptab = particles.astype(jnp.float32).reshape(N_NODES, D)
    ratios, rmw = _quat_edges_sc(ptab, ec, weights.astype(jnp.float32))
    return ratios.reshape(N_EDGES, P, 4), rmw.reshape(N_EDGES, P)
